# Initial kernel scaffold; baseline (speedup 1.0000x reference)
import jax
import jax.numpy as jnp
from jax.experimental import pallas as pl


def kernel(x, edge_index, lambda_max, l1w0, l1b0, l1w1, l1b1, l1w2, l1b2, l2w0, l2b0, l2w1, l2b1, l2w2, l2b2, l3w0, l3b0, l3w1, l3b1, l3w2, l3b2, l4w0, l4b0, l4w1, l4b1, l4w2, l4b2, fcw, fcb):
    raise NotImplementedError("write your pallas kernel here")


# speedup vs baseline: 1.7458x; 1.7458x over previous
import functools

import jax
import jax.numpy as jnp
from jax import lax
from jax.experimental import pallas as pl
from jax.experimental.pallas import tpu as pltpu
from jax.experimental.pallas import tpu_sc as plsc

N_NODES = 10000
NP = 10240
E_EDGES = 320000
ACC_ROWS = 10400
DUMMY_ROW = NP
KMAX = 6
N_TEC = 16
ROWS_PER_TEC = NP // N_TEC
ROW_TILE = 128
EDGE_B = 80
ZERO_TILE = 130


def _rsqrt_newton(t):
    i = plsc.bitcast(t, jnp.int32)
    y = plsc.bitcast(jnp.int32(0x5F3759DF) - lax.shift_right_arithmetic(i, 1),
                     jnp.float32)
    for _ in range(3):
        y = y * (1.5 - 0.5 * t * y * y)
    return y


_SC_PARAMS = pltpu.CompilerParams(needs_layout_passes=False,
                                  use_tc_tiling_on_sc=False)


def _setup_sc():
    mesh = plsc.VectorSubcoreMesh(core_axis_name="c", subcore_axis_name="s")
    EB = 400
    E_PER_W = E_EDGES // 32
    E_PER_T = E_EDGES // N_TEC

    @functools.partial(
        pl.kernel,
        out_type=[jax.ShapeDtypeStruct((E_EDGES,), jnp.int32),
                  jax.ShapeDtypeStruct((NP,), jnp.float32)],
        mesh=mesh,
        scratch_types=[
            pltpu.VMEM((EB,), jnp.int32),
            pltpu.VMEM((EB,), jnp.int32),
            pltpu.VMEM((EB,), jnp.int32),
            pltpu.VMEM((NP,), jnp.float32),
            pltpu.VMEM((ROWS_PER_TEC,), jnp.float32),
            pltpu.VMEM((ROWS_PER_TEC,), jnp.float32),
            pltpu.VMEM((16,), jnp.float32),
            pltpu.VMEM_SHARED((N_TEC, NP), jnp.float32),
        ],
        compiler_params=_SC_PARAMS,
    )
    def setup(row_h, col_h, lam_h, rowp_h, g_h,
              rbuf, cbuf, pbuf, degbuf, redbuf, sumbuf, lamv, deg16):
        cid = lax.axis_index("c")
        sid = lax.axis_index("s")
        wid = cid * N_TEC + sid

        def rowp_tile(t, _):
            e0 = wid * E_PER_W + t * EB
            pltpu.sync_copy(row_h.at[pl.ds(e0, EB)], rbuf)
            pltpu.sync_copy(col_h.at[pl.ds(e0, EB)], cbuf)
            for i in range(EB // 16):
                s = pl.ds(i * 16, 16)
                rv = rbuf[s]
                cv = cbuf[s]
                pbuf[s] = jnp.where(rv == cv, jnp.int32(DUMMY_ROW), rv)
            pltpu.sync_copy(pbuf, rowp_h.at[pl.ds(e0, EB)])
            return _
        lax.fori_loop(0, E_PER_W // EB, rowp_tile, 0)

        @pl.when(cid == 0)
        def _():
            def zero_deg(i, _):
                degbuf[pl.ds(i * 16, 16)] = jnp.zeros((16,), jnp.float32)
                return _
            lax.fori_loop(0, NP // 16, zero_deg, 0)

            def deg_tile(t, _):
                e0 = sid * E_PER_T + t * EB
                pltpu.sync_copy(row_h.at[pl.ds(e0, EB)], rbuf)
                pltpu.sync_copy(col_h.at[pl.ds(e0, EB)], cbuf)
                for i in range(EB // 16):
                    s = pl.ds(i * 16, 16)
                    rv = rbuf[s]
                    cv = cbuf[s]
                    w = jnp.where(rv == cv, jnp.float32(0.0), jnp.float32(1.0))
                    plsc.addupdate_scatter(degbuf, [rv], w)
                return _
            lax.fori_loop(0, E_PER_T // EB, deg_tile, 0)

            pltpu.sync_copy(degbuf, deg16.at[sid])
            plsc.subcore_barrier()

            base = sid * ROWS_PER_TEC
            def zs(i, _):
                sumbuf[pl.ds(i * 16, 16)] = jnp.zeros((16,), jnp.float32)
                return _
            lax.fori_loop(0, ROWS_PER_TEC // 16, zs, 0)
            def red(p, _):
                pltpu.sync_copy(deg16.at[p, pl.ds(base, ROWS_PER_TEC)], redbuf)
                def add(i, _):
                    s = pl.ds(i * 16, 16)
                    sumbuf[s] = sumbuf[s] + redbuf[s]
                    return _
                lax.fori_loop(0, ROWS_PER_TEC // 16, add, 0)
                return _
            lax.fori_loop(0, N_TEC, red, 0)

            pltpu.sync_copy(lam_h, lamv)
            def gcalc(i, _):
                s = pl.ds(i * 16, 16)
                dv = sumbuf[s]
                t = jnp.maximum(dv * lamv[...] * 0.5, jnp.float32(1e-30))
                y = _rsqrt_newton(t)
                sumbuf[s] = jnp.where(dv > 0.0, y, jnp.float32(0.0))
                return _
            lax.fori_loop(0, ROWS_PER_TEC // 16, gcalc, 0)
            pltpu.sync_copy(sumbuf, g_h.at[pl.ds(base, ROWS_PER_TEC)])

    return setup


def _layer_sc(C, W):
    mesh = plsc.VectorSubcoreMesh(core_axis_name="c", subcore_axis_name="s")
    CH = C // 2
    E_PER_T = E_EDGES // N_TEC
    NV = W // 16

    @functools.partial(
        pl.kernel,
        out_type=[jax.ShapeDtypeStruct((KMAX, C, NP, W), jnp.float32),
                  jax.ShapeDtypeStruct((C * NP, W), jnp.float32)],
        mesh=mesh,
        scratch_types=[
            pltpu.VMEM((ZERO_TILE, W), jnp.float32),
            pltpu.VMEM((ROW_TILE, W), jnp.float32),
            pltpu.VMEM((ROW_TILE, W), jnp.float32),
            pltpu.VMEM((ROW_TILE,), jnp.float32),
            pltpu.VMEM((EDGE_B,), jnp.int32),
            pltpu.VMEM((EDGE_B,), jnp.int32),
            pltpu.VMEM((16,), jnp.float32),
            pltpu.VMEM_SHARED((ACC_ROWS, W), jnp.float32),
            pltpu.SemaphoreType.DMA,
        ],
        compiler_params=_SC_PARAMS,
    )
    def layer(h_h, rowp_h, col_h, g_h, lam_h, tx_h, u_h,
              gbuf, t1buf, t0buf, gt, colv, rowv, lamv, acc, sem):
        cid = lax.axis_index("c")
        sid = lax.axis_index("s")
        pltpu.sync_copy(lam_h, lamv)
        d = (2.0 / lamv[...] - 1.0)[0]

        def chunk_body(j, _):
            c = cid * CH + j
            cbase = c * NP

            def ph_a(t, _):
                base = sid * ROWS_PER_TEC + t * ROW_TILE
                pltpu.sync_copy(h_h.at[c, pl.ds(base, ROW_TILE)], t0buf)
                pltpu.sync_copy(g_h.at[pl.ds(base, ROW_TILE)], gt)
                pltpu.sync_copy(t0buf, tx_h.at[0, c, pl.ds(base, ROW_TILE)])
                def rowfn(rb, _):
                    gvec = gt[pl.ds(rb * 16, 16)]
                    for i in range(16):
                        r = rb * 16 + i
                        gs = gvec[i]
                        for wv in range(NV):
                            s = pl.ds(wv * 16, 16)
                            t1buf[r, s] = gs * t0buf[r, s]
                    return _
                lax.fori_loop(0, ROW_TILE // 16, rowfn, 0)
                pltpu.sync_copy(t1buf, u_h.at[pl.ds(cbase + base, ROW_TILE)])
                return _
            lax.fori_loop(0, ROWS_PER_TEC // ROW_TILE, ph_a, 0)
            plsc.subcore_barrier()

            def k_body(k, _):
                ak = jnp.where(k == 1, jnp.float32(-1.0), jnp.float32(-2.0))
                bk = jnp.where(k == 1, d, 2.0 * d)
                ck = jnp.where(k == 1, jnp.float32(0.0), jnp.float32(1.0))
                km2 = jnp.maximum(k - 2, 0)

                def zb(r, _):
                    for wv in range(NV):
                        gbuf[r, pl.ds(wv * 16, 16)] = jnp.zeros((16,), jnp.float32)
                    return _
                lax.fori_loop(0, ZERO_TILE, zb, 0)
                def zt(t, _):
                    a0 = sid * (5 * ZERO_TILE) + t * ZERO_TILE
                    pltpu.sync_copy(gbuf, acc.at[pl.ds(a0, ZERO_TILE)])
                    return _
                lax.fori_loop(0, 5, zt, 0)
                plsc.subcore_barrier()

                def et(t, _):
                    e0 = sid * E_PER_T + t * EDGE_B
                    pltpu.sync_copy(col_h.at[pl.ds(e0, EDGE_B)], colv)
                    pltpu.sync_copy(rowp_h.at[pl.ds(e0, EDGE_B)], rowv)
                    for i in range(EDGE_B // 16):
                        s = pl.ds(i * 16, 16)
                        colv[s] = colv[s] + cbase
                    pltpu.async_copy(u_h.at[colv],
                                     gbuf.at[pl.ds(0, EDGE_B)], sem).wait()
                    pltpu.sync_copy(gbuf.at[pl.ds(0, EDGE_B)],
                                    acc.at[rowv], add=True)
                    return _
                lax.fori_loop(0, E_PER_T // EDGE_B, et, 0)
                plsc.subcore_barrier()

                def ep(t, carry):
                    base = sid * ROWS_PER_TEC + t * ROW_TILE
                    pltpu.sync_copy(acc.at[pl.ds(base, ROW_TILE)],
                                    gbuf.at[pl.ds(0, ROW_TILE)])
                    pltpu.sync_copy(tx_h.at[k - 1, c, pl.ds(base, ROW_TILE)],
                                    t1buf)
                    pltpu.sync_copy(tx_h.at[km2, c, pl.ds(base, ROW_TILE)],
                                    t0buf)
                    pltpu.sync_copy(g_h.at[pl.ds(base, ROW_TILE)], gt)
                    def rowfn(rb, _):
                        gvec = gt[pl.ds(rb * 16, 16)]
                        for i in range(16):
                            r = rb * 16 + i
                            gs = gvec[i]
                            for wv in range(NV):
                                s = pl.ds(wv * 16, 16)
                                v = ((ak * gs) * gbuf[r, s]
                                     + bk * t1buf[r, s] - ck * t0buf[r, s])
                                gbuf[r, s] = v
                                t1buf[r, s] = gs * v
                        return _
                    lax.fori_loop(0, ROW_TILE // 16, rowfn, 0)
                    pltpu.sync_copy(gbuf.at[pl.ds(0, ROW_TILE)],
                                    tx_h.at[k, c, pl.ds(base, ROW_TILE)])
                    @pl.when(k < KMAX - 1)
                    def _uw():
                        pltpu.sync_copy(
                            t1buf, u_h.at[pl.ds(cbase + base, ROW_TILE)])
                    return carry
                lax.fori_loop(0, ROWS_PER_TEC // ROW_TILE, ep, 0)
                plsc.subcore_barrier()
                return _
            lax.fori_loop(1, KMAX, k_body, 0)
            return _
        lax.fori_loop(0, CH, chunk_body, 0)

    return layer


def _matmul_tc(C_in, W_in, dout3, relu):
    M = 512
    KC = KMAX * C_in
    grid = (NP // M, KC)

    def mm(tx_ref, w_ref, b_ref, o_ref, accs):
        kci = pl.program_id(1)

        @pl.when(kci == 0)
        def _():
            accs[...] = jnp.zeros_like(accs)

        accs[...] += jnp.dot(tx_ref[0, 0], w_ref[0],
                             preferred_element_type=jnp.float32)

        @pl.when(kci == KC - 1)
        def _():
            r = accs[...] + b_ref[...]
            if relu:
                r = jnp.maximum(r, 0.0)
            o_ref[...] = r

    return pl.pallas_call(
        mm,
        grid=grid,
        in_specs=[
            pl.BlockSpec((1, 1, M, W_in),
                         lambda n, kci: (kci // C_in, kci % C_in, n, 0)),
            pl.BlockSpec((1, W_in, dout3), lambda n, kci: (kci, 0, 0)),
            pl.BlockSpec((1, dout3), lambda n, kci: (0, 0)),
        ],
        out_specs=pl.BlockSpec((M, dout3), lambda n, kci: (n, 0)),
        out_shape=jax.ShapeDtypeStruct((NP, dout3), jnp.float32),
        scratch_shapes=[pltpu.VMEM((M, dout3), jnp.float32)],
        compiler_params=pltpu.CompilerParams(
            dimension_semantics=("parallel", "arbitrary")),
    )


def _pack_weights(ws, bs, C_in, W_in):
    din = C_in * W_in
    dout = ws[0].shape[-1]
    dout3 = 3 * dout
    wf = jnp.zeros((KMAX, din, dout3), jnp.float32)
    for s, w in enumerate(ws):
        k = w.shape[0]
        wf = wf.at[:k, :, s * dout:(s + 1) * dout].set(w)
    wf = wf.reshape(KMAX * C_in, W_in, dout3)
    bf = jnp.concatenate(bs).reshape(1, dout3)
    return wf, bf


_CFG = [(2, 64), (4, 48), (6, 64), (12, 64)]


def kernel(x, edge_index, lambda_max,
           l1w0, l1b0, l1w1, l1b1, l1w2, l1b2,
           l2w0, l2b0, l2w1, l2b1, l2w2, l2b2,
           l3w0, l3b0, l3w1, l3b1, l3w2, l3b2,
           l4w0, l4b0, l4w1, l4b1, l4w2, l4b2,
           fcw, fcb):
    row = edge_index[0]
    col = edge_index[1]
    lam16 = jnp.broadcast_to(lambda_max.astype(jnp.float32), (16,))

    rowp, g = _setup_sc()(row, col, lam16)

    h = jnp.zeros((NP, 128), jnp.float32).at[:N_NODES].set(x)

    wss = [(l1w0, l1w1, l1w2), (l2w0, l2w1, l2w2),
           (l3w0, l3w1, l3w2), (l4w0, l4w1, l4w2)]
    bss = [(l1b0, l1b1, l1b2), (l2b0, l2b1, l2b2),
           (l3b0, l3b1, l3b2), (l4b0, l4b1, l4b2)]

    for li in range(4):
        C_in, W_in = _CFG[li]
        hc = h.reshape(NP, C_in, W_in).transpose(1, 0, 2)
        tx, _u = _layer_sc(C_in, W_in)(hc, rowp, col, g, lam16)
        wf, bf = _pack_weights(wss[li], bss[li], C_in, W_in)
        dout3 = bf.shape[-1]
        h = _matmul_tc(C_in, W_in, dout3, relu=True)(tx, wf, bf)

    M = 512

    def fc(h_ref, w_ref, b_ref, o_ref):
        o_ref[...] = jnp.dot(h_ref[...], w_ref[...],
                             preferred_element_type=jnp.float32) + b_ref[...]

    out = pl.pallas_call(
        fc,
        grid=(NP // M,),
        in_specs=[
            pl.BlockSpec((M, 768), lambda n: (n, 0)),
            pl.BlockSpec((768, 16), lambda n: (0, 0)),
            pl.BlockSpec((1, 16), lambda n: (0, 0)),
        ],
        out_specs=pl.BlockSpec((M, 16), lambda n: (n, 0)),
        out_shape=jax.ShapeDtypeStruct((NP, 16), jnp.float32),
        compiler_params=pltpu.CompilerParams(
            dimension_semantics=("parallel",)),
    )(h, fcw, fcb.reshape(1, 16))

    return out[:N_NODES]


# speedup vs baseline: 4.9003x; 2.8069x over previous
import functools

import jax
import jax.numpy as jnp
from jax import lax
from jax.experimental import pallas as pl
from jax.experimental.pallas import tpu as pltpu
from jax.experimental.pallas import tpu_sc as plsc

N_NODES = 10000
NP = 10240
E_EDGES = 320000
ACC_ROWS = 10256
DUMMY_ROW = NP
KMAX = 6
N_TEC = 16
ROWS_PER_TEC = NP // N_TEC
ROW_TILE = 80
EDGE_B = 80


def _rsqrt_newton(t):
    i = plsc.bitcast(t, jnp.int32)
    y = plsc.bitcast(jnp.int32(0x5F3759DF) - lax.shift_right_arithmetic(i, 1),
                     jnp.float32)
    for _ in range(3):
        y = y * (1.5 - 0.5 * t * y * y)
    return y


_SC_PARAMS = pltpu.CompilerParams(needs_layout_passes=False,
                                  use_tc_tiling_on_sc=False)


def _setup_sc():
    mesh = plsc.VectorSubcoreMesh(core_axis_name="c", subcore_axis_name="s")
    EB = 400
    E_PER_W = E_EDGES // 32
    E_PER_T = E_EDGES // N_TEC

    @functools.partial(
        pl.kernel,
        out_type=[jax.ShapeDtypeStruct((E_EDGES // EDGE_B, 1, EDGE_B), jnp.int32),
                  jax.ShapeDtypeStruct((NP,), jnp.float32)],
        mesh=mesh,
        scratch_types=[
            pltpu.VMEM((EB,), jnp.int32),
            pltpu.VMEM((EB,), jnp.int32),
            pltpu.VMEM((EB // EDGE_B, 1, EDGE_B), jnp.int32),
            pltpu.VMEM((NP,), jnp.float32),
            pltpu.VMEM((ROWS_PER_TEC,), jnp.float32),
            pltpu.VMEM((ROWS_PER_TEC,), jnp.float32),
            pltpu.VMEM((16,), jnp.float32),
            pltpu.VMEM_SHARED((N_TEC, NP), jnp.float32),
        ],
        compiler_params=_SC_PARAMS,
    )
    def setup(row_h, col_h, lam_h, rowp_h, g_h,
              rbuf, cbuf, pbuf, degbuf, redbuf, sumbuf, lamv, deg16):
        cid = lax.axis_index("c")
        sid = lax.axis_index("s")
        wid = cid * N_TEC + sid

        def rowp_tile(t, _):
            e0 = wid * E_PER_W + t * EB
            pltpu.sync_copy(row_h.at[pl.ds(e0, EB)], rbuf)
            pltpu.sync_copy(col_h.at[pl.ds(e0, EB)], cbuf)
            for i in range(EB // 16):
                s = pl.ds(i * 16, 16)
                rv = rbuf[s]
                cv = cbuf[s]
                pbuf[i // 5, 0, pl.ds((i % 5) * 16, 16)] = jnp.where(
                    rv == cv, jnp.int32(DUMMY_ROW), rv)
            pltpu.sync_copy(pbuf, rowp_h.at[pl.ds(e0 // EDGE_B, EB // EDGE_B)])
            return _
        lax.fori_loop(0, E_PER_W // EB, rowp_tile, 0)

        @pl.when(cid == 0)
        def _():
            def zero_deg(i, _):
                degbuf[pl.ds(i * 16, 16)] = jnp.zeros((16,), jnp.float32)
                return _
            lax.fori_loop(0, NP // 16, zero_deg, 0)

            def deg_tile(t, _):
                e0 = sid * E_PER_T + t * EB
                pltpu.sync_copy(row_h.at[pl.ds(e0, EB)], rbuf)
                pltpu.sync_copy(col_h.at[pl.ds(e0, EB)], cbuf)
                for i in range(EB // 16):
                    s = pl.ds(i * 16, 16)
                    rv = rbuf[s]
                    cv = cbuf[s]
                    w = jnp.where(rv == cv, jnp.float32(0.0), jnp.float32(1.0))
                    plsc.addupdate_scatter(degbuf, [rv], w)
                return _
            lax.fori_loop(0, E_PER_T // EB, deg_tile, 0)

            pltpu.sync_copy(degbuf, deg16.at[sid])
            plsc.subcore_barrier()

            base = sid * ROWS_PER_TEC
            def zs(i, _):
                sumbuf[pl.ds(i * 16, 16)] = jnp.zeros((16,), jnp.float32)
                return _
            lax.fori_loop(0, ROWS_PER_TEC // 16, zs, 0)
            def red(p, _):
                pltpu.sync_copy(deg16.at[p, pl.ds(base, ROWS_PER_TEC)], redbuf)
                def add(i, _):
                    s = pl.ds(i * 16, 16)
                    sumbuf[s] = sumbuf[s] + redbuf[s]
                    return _
                lax.fori_loop(0, ROWS_PER_TEC // 16, add, 0)
                return _
            lax.fori_loop(0, N_TEC, red, 0)

            pltpu.sync_copy(lam_h, lamv)
            def gcalc(i, _):
                s = pl.ds(i * 16, 16)
                dv = sumbuf[s]
                t = jnp.maximum(dv * lamv[...] * 0.5, jnp.float32(1e-30))
                y = _rsqrt_newton(t)
                sumbuf[s] = jnp.where(dv > 0.0, y, jnp.float32(0.0))
                return _
            lax.fori_loop(0, ROWS_PER_TEC // 16, gcalc, 0)
            pltpu.sync_copy(sumbuf, g_h.at[pl.ds(base, ROWS_PER_TEC)])

    return setup


def _layer_sc(C, W):
    mesh = plsc.VectorSubcoreMesh(core_axis_name="c", subcore_axis_name="s")
    CH = C // 2
    E_PER_T = E_EDGES // N_TEC
    NV = W // 16
    TILES = E_PER_T // EDGE_B
    NBUF = 3
    GRPS = TILES // NBUF
    TAIL = TILES - GRPS * NBUF

    @functools.partial(
        pl.kernel,
        out_type=[jax.ShapeDtypeStruct((KMAX, C, NP, W), jnp.float32),
                  jax.ShapeDtypeStruct((C * NP, W), jnp.float32)],
        mesh=mesh,
        scratch_types=[
            pltpu.VMEM((NBUF, EDGE_B, W), jnp.float32),
            pltpu.VMEM((ROW_TILE,), jnp.float32),
            pltpu.VMEM((TILES, 1, EDGE_B), jnp.int32),
            pltpu.VMEM((TILES, 1, EDGE_B), jnp.int32),
            pltpu.VMEM((16,), jnp.float32),
            pltpu.VMEM_SHARED((ACC_ROWS, W), jnp.float32),
        ] + [pltpu.SemaphoreType.DMA] * (2 * NBUF),
        compiler_params=_SC_PARAMS,
    )
    def layer(h_h, rowp_h, col_h, g_h, lam_h, tx_h, u_h,
              ring, gt, colv, rowv, lamv, acc, *sems):
        sem_g = sems[:NBUF]
        sem_s = sems[NBUF:]
        cid = lax.axis_index("c")
        sid = lax.axis_index("s")
        pltpu.sync_copy(lam_h, lamv)
        d = (2.0 / lamv[...] - 1.0)[0]
        pltpu.sync_copy(rowp_h.at[pl.ds(sid * TILES, TILES)], rowv)

        def chunk_body(j, _):
            c = cid * CH + j
            cbase = c * NP

            pltpu.sync_copy(col_h.at[pl.ds(sid * TILES, TILES)], colv)
            def cadd(r, carry):
                for i in range(EDGE_B // 16):
                    s = pl.ds(i * 16, 16)
                    colv[r, 0, s] = colv[r, 0, s] + cbase
                return carry
            lax.fori_loop(0, TILES, cadd, 0)

            def ph_a(t, carry):
                base = sid * ROWS_PER_TEC + t * ROW_TILE
                r0 = ring.at[0]
                r1 = ring.at[1]
                pltpu.sync_copy(h_h.at[c, pl.ds(base, ROW_TILE)], r0)
                pltpu.sync_copy(g_h.at[pl.ds(base, ROW_TILE)], gt)
                pltpu.sync_copy(r0, tx_h.at[0, c, pl.ds(base, ROW_TILE)])
                def rowfn(rb, cy):
                    gvec = gt[pl.ds(rb * 16, 16)]
                    for i in range(16):
                        r = rb * 16 + i
                        gs = gvec[i]
                        for wv in range(NV):
                            s = pl.ds(wv * 16, 16)
                            r1[r, s] = gs * r0[r, s]
                    return cy
                lax.fori_loop(0, ROW_TILE // 16, rowfn, 0)
                pltpu.sync_copy(r1, u_h.at[pl.ds(cbase + base, ROW_TILE)])
                return carry
            lax.fori_loop(0, ROWS_PER_TEC // ROW_TILE, ph_a, 0)
            plsc.subcore_barrier()

            def k_body(k, _):
                ak = jnp.where(k == 1, jnp.float32(-1.0), jnp.float32(-2.0))
                bk = jnp.where(k == 1, d, 2.0 * d)
                ck = jnp.where(k == 1, jnp.float32(0.0), jnp.float32(1.0))
                km2 = jnp.maximum(k - 2, 0)

                def zb(r, cy):
                    for wv in range(NV):
                        ring[0, r, pl.ds(wv * 16, 16)] = jnp.zeros(
                            (16,), jnp.float32)
                    return cy
                lax.fori_loop(0, EDGE_B, zb, 0)
                def zt(t, cy):
                    a0 = sid * ROWS_PER_TEC + t * EDGE_B
                    pltpu.sync_copy(ring.at[0], acc.at[pl.ds(a0, EDGE_B)])
                    return cy
                lax.fori_loop(0, ROWS_PER_TEC // EDGE_B, zt, 0)
                @pl.when(sid == 0)
                def _zd():
                    pltpu.sync_copy(ring.at[0, pl.ds(0, ACC_ROWS - NP)],
                                    acc.at[pl.ds(NP, ACC_ROWS - NP)])
                plsc.subcore_barrier()

                def egrp(grp, carry):
                    t0i = grp * NBUF
                    ghs = [pltpu.async_copy(
                        u_h.at[colv.at[t0i + b, 0]], ring.at[b], sem_g[b])
                        for b in range(NBUF)]
                    shs = []
                    for b in range(NBUF):
                        ghs[b].wait()
                        shs.append(pltpu.async_copy(
                            ring.at[b], acc.at[rowv.at[t0i + b, 0]],
                            sem_s[b], add=True))
                    for b in range(NBUF):
                        shs[b].wait()
                    return carry
                lax.fori_loop(0, GRPS, egrp, 0)
                for b in range(TAIL):
                    t0i = GRPS * NBUF + b
                    pltpu.async_copy(u_h.at[colv.at[t0i, 0]], ring.at[b],
                                     sem_g[b]).wait()
                    pltpu.async_copy(ring.at[b], acc.at[rowv.at[t0i, 0]],
                                     sem_s[b], add=True).wait()
                plsc.subcore_barrier()

                def ep(t, carry):
                    base = sid * ROWS_PER_TEC + t * ROW_TILE
                    r0 = ring.at[0]
                    r1 = ring.at[1]
                    r2 = ring.at[2]
                    pltpu.sync_copy(acc.at[pl.ds(base, ROW_TILE)], r0)
                    pltpu.sync_copy(tx_h.at[k - 1, c, pl.ds(base, ROW_TILE)], r2)
                    pltpu.sync_copy(tx_h.at[km2, c, pl.ds(base, ROW_TILE)], r1)
                    pltpu.sync_copy(g_h.at[pl.ds(base, ROW_TILE)], gt)
                    def rowfn(rb, cy):
                        gvec = gt[pl.ds(rb * 16, 16)]
                        for i in range(16):
                            r = rb * 16 + i
                            gs = gvec[i]
                            for wv in range(NV):
                                s = pl.ds(wv * 16, 16)
                                v = ((ak * gs) * r0[r, s]
                                     + bk * r2[r, s] - ck * r1[r, s])
                                r0[r, s] = v
                                r1[r, s] = gs * v
                        return cy
                    lax.fori_loop(0, ROW_TILE // 16, rowfn, 0)
                    pltpu.sync_copy(r0, tx_h.at[k, c, pl.ds(base, ROW_TILE)])
                    @pl.when(k < KMAX - 1)
                    def _uw():
                        pltpu.sync_copy(r1, u_h.at[pl.ds(cbase + base, ROW_TILE)])
                    return carry
                lax.fori_loop(0, ROWS_PER_TEC // ROW_TILE, ep, 0)
                plsc.subcore_barrier()
                return _
            lax.fori_loop(1, KMAX, k_body, 0)
            return _
        lax.fori_loop(0, CH, chunk_body, 0)

    return layer


def _matmul_tc(C_in, W_in, dout3, relu):
    M = 512
    KC = KMAX * C_in
    grid = (NP // M, KC)

    def mm(tx_ref, w_ref, b_ref, o_ref, accs):
        kci = pl.program_id(1)

        @pl.when(kci == 0)
        def _():
            accs[...] = jnp.zeros_like(accs)

        accs[...] += jnp.dot(tx_ref[0, 0], w_ref[0],
                             preferred_element_type=jnp.float32)

        @pl.when(kci == KC - 1)
        def _():
            r = accs[...] + b_ref[...]
            if relu:
                r = jnp.maximum(r, 0.0)
            o_ref[...] = r

    return pl.pallas_call(
        mm,
        grid=grid,
        in_specs=[
            pl.BlockSpec((1, 1, M, W_in),
                         lambda n, kci: (kci // C_in, kci % C_in, n, 0)),
            pl.BlockSpec((1, W_in, dout3), lambda n, kci: (kci, 0, 0)),
            pl.BlockSpec((1, dout3), lambda n, kci: (0, 0)),
        ],
        out_specs=pl.BlockSpec((M, dout3), lambda n, kci: (n, 0)),
        out_shape=jax.ShapeDtypeStruct((NP, dout3), jnp.float32),
        scratch_shapes=[pltpu.VMEM((M, dout3), jnp.float32)],
        compiler_params=pltpu.CompilerParams(
            dimension_semantics=("parallel", "arbitrary")),
    )


def _pack_weights(ws, bs, C_in, W_in):
    din = C_in * W_in
    dout = ws[0].shape[-1]
    dout3 = 3 * dout
    wf = jnp.zeros((KMAX, din, dout3), jnp.float32)
    for s, w in enumerate(ws):
        k = w.shape[0]
        wf = wf.at[:k, :, s * dout:(s + 1) * dout].set(w)
    wf = wf.reshape(KMAX * C_in, W_in, dout3)
    bf = jnp.concatenate(bs).reshape(1, dout3)
    return wf, bf


_CFG = [(2, 64), (2, 96), (4, 96), (8, 96)]


def kernel(x, edge_index, lambda_max,
           l1w0, l1b0, l1w1, l1b1, l1w2, l1b2,
           l2w0, l2b0, l2w1, l2b1, l2w2, l2b2,
           l3w0, l3b0, l3w1, l3b1, l3w2, l3b2,
           l4w0, l4b0, l4w1, l4b1, l4w2, l4b2,
           fcw, fcb):
    row = edge_index[0]
    col = edge_index[1]
    lam16 = jnp.broadcast_to(lambda_max.astype(jnp.float32), (16,))

    rowp, g = _setup_sc()(row, col, lam16)
    col3 = col.reshape(E_EDGES // EDGE_B, 1, EDGE_B)

    h = jnp.zeros((NP, 128), jnp.float32).at[:N_NODES].set(x)

    wss = [(l1w0, l1w1, l1w2), (l2w0, l2w1, l2w2),
           (l3w0, l3w1, l3w2), (l4w0, l4w1, l4w2)]
    bss = [(l1b0, l1b1, l1b2), (l2b0, l2b1, l2b2),
           (l3b0, l3b1, l3b2), (l4b0, l4b1, l4b2)]

    for li in range(4):
        C_in, W_in = _CFG[li]
        hc = h.reshape(NP, C_in, W_in).transpose(1, 0, 2)
        tx, _u = _layer_sc(C_in, W_in)(hc, rowp, col3, g, lam16)
        wf, bf = _pack_weights(wss[li], bss[li], C_in, W_in)
        dout3 = bf.shape[-1]
        h = _matmul_tc(C_in, W_in, dout3, relu=True)(tx, wf, bf)

    M = 512

    def fc(h_ref, w_ref, b_ref, o_ref):
        o_ref[...] = jnp.dot(h_ref[...], w_ref[...],
                             preferred_element_type=jnp.float32) + b_ref[...]

    out = pl.pallas_call(
        fc,
        grid=(NP // M,),
        in_specs=[
            pl.BlockSpec((M, 768), lambda n: (n, 0)),
            pl.BlockSpec((768, 16), lambda n: (0, 0)),
            pl.BlockSpec((1, 16), lambda n: (0, 0)),
        ],
        out_specs=pl.BlockSpec((M, 16), lambda n: (n, 0)),
        out_shape=jax.ShapeDtypeStruct((NP, 16), jnp.float32),
        compiler_params=pltpu.CompilerParams(
            dimension_semantics=("parallel",)),
    )(h, fcw, fcb.reshape(1, 16))

    return out[:N_NODES]


# speedup vs baseline: 5.9554x; 1.2153x over previous
import functools

import jax
import jax.numpy as jnp
from jax import lax
from jax.experimental import pallas as pl
from jax.experimental.pallas import tpu as pltpu
from jax.experimental.pallas import tpu_sc as plsc

N_NODES = 10000
NP = 10240
E_EDGES = 320000
ACC_ROWS = 10256
DUMMY_ROW = NP
KMAX = 6
N_TEC = 16
ROWS_PER_TEC = NP // N_TEC
ROW_TILE = 80
EDGE_B = 80


def _rsqrt_newton(t):
    i = plsc.bitcast(t, jnp.int32)
    y = plsc.bitcast(jnp.int32(0x5F3759DF) - lax.shift_right_arithmetic(i, 1),
                     jnp.float32)
    for _ in range(3):
        y = y * (1.5 - 0.5 * t * y * y)
    return y


_SC_PARAMS = pltpu.CompilerParams(needs_layout_passes=False,
                                  use_tc_tiling_on_sc=False)


def _setup_sc():
    mesh = plsc.VectorSubcoreMesh(core_axis_name="c", subcore_axis_name="s")
    EB = 400
    E_PER_W = E_EDGES // 32
    E_PER_T = E_EDGES // N_TEC

    @functools.partial(
        pl.kernel,
        out_type=[jax.ShapeDtypeStruct((E_EDGES // EDGE_B, 1, EDGE_B), jnp.int32),
                  jax.ShapeDtypeStruct((NP,), jnp.float32)],
        mesh=mesh,
        scratch_types=[
            pltpu.VMEM((EB,), jnp.int32),
            pltpu.VMEM((EB,), jnp.int32),
            pltpu.VMEM((EB // EDGE_B, 1, EDGE_B), jnp.int32),
            pltpu.VMEM((NP,), jnp.float32),
            pltpu.VMEM((ROWS_PER_TEC,), jnp.float32),
            pltpu.VMEM((ROWS_PER_TEC,), jnp.float32),
            pltpu.VMEM((16,), jnp.float32),
            pltpu.VMEM_SHARED((N_TEC, NP), jnp.float32),
        ],
        compiler_params=_SC_PARAMS,
    )
    def setup(row_h, col_h, lam_h, rowp_h, g_h,
              rbuf, cbuf, pbuf, degbuf, redbuf, sumbuf, lamv, deg16):
        cid = lax.axis_index("c")
        sid = lax.axis_index("s")
        wid = cid * N_TEC + sid

        def rowp_tile(t, _):
            e0 = wid * E_PER_W + t * EB
            pltpu.sync_copy(row_h.at[pl.ds(e0, EB)], rbuf)
            pltpu.sync_copy(col_h.at[pl.ds(e0, EB)], cbuf)
            for i in range(EB // 16):
                s = pl.ds(i * 16, 16)
                rv = rbuf[s]
                cv = cbuf[s]
                pbuf[i // 5, 0, pl.ds((i % 5) * 16, 16)] = jnp.where(
                    rv == cv, jnp.int32(DUMMY_ROW), rv)
            pltpu.sync_copy(pbuf, rowp_h.at[pl.ds(e0 // EDGE_B, EB // EDGE_B)])
            return _
        lax.fori_loop(0, E_PER_W // EB, rowp_tile, 0)

        @pl.when(cid == 0)
        def _():
            def zero_deg(i, _):
                degbuf[pl.ds(i * 16, 16)] = jnp.zeros((16,), jnp.float32)
                return _
            lax.fori_loop(0, NP // 16, zero_deg, 0)

            def deg_tile(t, _):
                e0 = sid * E_PER_T + t * EB
                pltpu.sync_copy(row_h.at[pl.ds(e0, EB)], rbuf)
                pltpu.sync_copy(col_h.at[pl.ds(e0, EB)], cbuf)
                for i in range(EB // 16):
                    s = pl.ds(i * 16, 16)
                    rv = rbuf[s]
                    cv = cbuf[s]
                    w = jnp.where(rv == cv, jnp.float32(0.0), jnp.float32(1.0))
                    plsc.addupdate_scatter(degbuf, [rv], w)
                return _
            lax.fori_loop(0, E_PER_T // EB, deg_tile, 0)

            pltpu.sync_copy(degbuf, deg16.at[sid])
            plsc.subcore_barrier()

            base = sid * ROWS_PER_TEC
            def zs(i, _):
                sumbuf[pl.ds(i * 16, 16)] = jnp.zeros((16,), jnp.float32)
                return _
            lax.fori_loop(0, ROWS_PER_TEC // 16, zs, 0)
            def red(p, _):
                pltpu.sync_copy(deg16.at[p, pl.ds(base, ROWS_PER_TEC)], redbuf)
                def add(i, _):
                    s = pl.ds(i * 16, 16)
                    sumbuf[s] = sumbuf[s] + redbuf[s]
                    return _
                lax.fori_loop(0, ROWS_PER_TEC // 16, add, 0)
                return _
            lax.fori_loop(0, N_TEC, red, 0)

            pltpu.sync_copy(lam_h, lamv)
            def gcalc(i, _):
                s = pl.ds(i * 16, 16)
                dv = sumbuf[s]
                t = jnp.maximum(dv * lamv[...] * 0.5, jnp.float32(1e-30))
                y = _rsqrt_newton(t)
                sumbuf[s] = jnp.where(dv > 0.0, y, jnp.float32(0.0))
                return _
            lax.fori_loop(0, ROWS_PER_TEC // 16, gcalc, 0)
            pltpu.sync_copy(sumbuf, g_h.at[pl.ds(base, ROWS_PER_TEC)])

    return setup


def _layer_sc(C, W):
    mesh = plsc.VectorSubcoreMesh(core_axis_name="c", subcore_axis_name="s")
    CH = C // 2
    E_PER_T = E_EDGES // N_TEC
    NV = W // 16
    TILES = E_PER_T // EDGE_B
    NBUF = 3
    GRPS = TILES // NBUF
    TAIL = TILES - GRPS * NBUF

    @functools.partial(
        pl.kernel,
        out_type=[jax.ShapeDtypeStruct((KMAX, C, NP, W), jnp.float32),
                  jax.ShapeDtypeStruct((C * NP, W), jnp.float32)],
        mesh=mesh,
        scratch_types=[
            pltpu.VMEM((NBUF, EDGE_B, W), jnp.float32),
            pltpu.VMEM((ROW_TILE,), jnp.float32),
            pltpu.VMEM((TILES, 1, EDGE_B), jnp.int32),
            pltpu.VMEM((TILES, 1, EDGE_B), jnp.int32),
            pltpu.VMEM((16,), jnp.float32),
            pltpu.VMEM_SHARED((ACC_ROWS, W), jnp.float32),
        ] + [pltpu.SemaphoreType.DMA] * (2 * NBUF),
        compiler_params=_SC_PARAMS,
    )
    def layer(h_h, rowp_h, col_h, g_h, lam_h, tx_h, u_h,
              ring, gt, colv, rowv, lamv, acc, *sems):
        sem_g = sems[:NBUF]
        sem_s = sems[NBUF:]
        cid = lax.axis_index("c")
        sid = lax.axis_index("s")
        pltpu.sync_copy(lam_h, lamv)
        d = (2.0 / lamv[...] - 1.0)[0]
        pltpu.sync_copy(rowp_h.at[pl.ds(sid * TILES, TILES)], rowv)

        def chunk_body(j, _):
            c = cid * CH + j
            cbase = c * NP

            pltpu.sync_copy(col_h.at[pl.ds(sid * TILES, TILES)], colv)
            def cadd(r, carry):
                for i in range(EDGE_B // 16):
                    s = pl.ds(i * 16, 16)
                    colv[r, 0, s] = colv[r, 0, s] + cbase
                return carry
            lax.fori_loop(0, TILES, cadd, 0)

            def ph_a(t, carry):
                base = sid * ROWS_PER_TEC + t * ROW_TILE
                r0 = ring.at[0]
                r1 = ring.at[1]
                pltpu.sync_copy(h_h.at[c, pl.ds(base, ROW_TILE)], r0)
                pltpu.sync_copy(g_h.at[pl.ds(base, ROW_TILE)], gt)
                pltpu.sync_copy(r0, tx_h.at[0, c, pl.ds(base, ROW_TILE)])
                def rowfn(rb, cy):
                    gvec = gt[pl.ds(rb * 16, 16)]
                    for i in range(16):
                        r = rb * 16 + i
                        gs = gvec[i]
                        for wv in range(NV):
                            s = pl.ds(wv * 16, 16)
                            r1[r, s] = gs * r0[r, s]
                    return cy
                lax.fori_loop(0, ROW_TILE // 16, rowfn, 0)
                pltpu.sync_copy(r1, u_h.at[pl.ds(cbase + base, ROW_TILE)])
                return carry
            lax.fori_loop(0, ROWS_PER_TEC // ROW_TILE, ph_a, 0)
            plsc.subcore_barrier()

            def k_body(k, _):
                ak = jnp.where(k == 1, jnp.float32(-1.0), jnp.float32(-2.0))
                bk = jnp.where(k == 1, d, 2.0 * d)
                ck = jnp.where(k == 1, jnp.float32(0.0), jnp.float32(1.0))
                km2 = jnp.maximum(k - 2, 0)

                def zb(r, cy):
                    for wv in range(NV):
                        ring[0, r, pl.ds(wv * 16, 16)] = jnp.zeros(
                            (16,), jnp.float32)
                    return cy
                lax.fori_loop(0, EDGE_B, zb, 0)
                zhs = []
                for t in range(ROWS_PER_TEC // EDGE_B):
                    a0 = sid * ROWS_PER_TEC + t * EDGE_B
                    zhs.append(pltpu.async_copy(
                        ring.at[0], acc.at[pl.ds(a0, EDGE_B)], sem_g[0]))
                @pl.when(sid == 0)
                def _zd():
                    pltpu.sync_copy(ring.at[0, pl.ds(0, ACC_ROWS - NP)],
                                    acc.at[pl.ds(NP, ACC_ROWS - NP)])
                for h in zhs:
                    h.wait()
                plsc.subcore_barrier()

                for b in range(NBUF):
                    pltpu.async_copy(u_h.at[colv.at[b, 0]], ring.at[b],
                                     sem_g[b])
                def egrp(grp, carry):
                    t0i = grp * NBUF
                    for b in range(NBUF):
                        pltpu.make_async_copy(
                            u_h.at[colv.at[t0i + b, 0]], ring.at[b],
                            sem_g[b]).wait()
                        pltpu.async_copy(
                            ring.at[b], acc.at[rowv.at[t0i + b, 0]],
                            sem_s[b], add=True)
                    for b in range(NBUF):
                        pltpu.make_async_copy(
                            ring.at[b], acc.at[rowv.at[t0i + b, 0]],
                            sem_s[b]).wait()
                        @pl.when(grp < GRPS - 1)
                        def _nx():
                            pltpu.async_copy(
                                u_h.at[colv.at[t0i + NBUF + b, 0]],
                                ring.at[b], sem_g[b])
                    return carry
                lax.fori_loop(0, GRPS, egrp, 0)
                for b in range(TAIL):
                    t0i = GRPS * NBUF + b
                    pltpu.async_copy(u_h.at[colv.at[t0i, 0]], ring.at[b],
                                     sem_g[b]).wait()
                    pltpu.async_copy(ring.at[b], acc.at[rowv.at[t0i, 0]],
                                     sem_s[b], add=True).wait()
                plsc.subcore_barrier()

                def ep(t, carry):
                    base = sid * ROWS_PER_TEC + t * ROW_TILE
                    r0 = ring.at[0]
                    r1 = ring.at[1]
                    r2 = ring.at[2]
                    lh = [
                        pltpu.async_copy(acc.at[pl.ds(base, ROW_TILE)], r0,
                                         sem_g[0]),
                        pltpu.async_copy(
                            tx_h.at[k - 1, c, pl.ds(base, ROW_TILE)], r2,
                            sem_g[1]),
                        pltpu.async_copy(
                            tx_h.at[km2, c, pl.ds(base, ROW_TILE)], r1,
                            sem_g[2]),
                    ]
                    pltpu.sync_copy(g_h.at[pl.ds(base, ROW_TILE)], gt)
                    for h in lh:
                        h.wait()
                    def rowfn(rb, cy):
                        gvec = gt[pl.ds(rb * 16, 16)]
                        for i in range(16):
                            r = rb * 16 + i
                            gs = gvec[i]
                            for wv in range(NV):
                                s = pl.ds(wv * 16, 16)
                                v = ((ak * gs) * r0[r, s]
                                     + bk * r2[r, s] - ck * r1[r, s])
                                r0[r, s] = v
                                r1[r, s] = gs * v
                        return cy
                    lax.fori_loop(0, ROW_TILE // 16, rowfn, 0)
                    pltpu.sync_copy(r0, tx_h.at[k, c, pl.ds(base, ROW_TILE)])
                    @pl.when(k < KMAX - 1)
                    def _uw():
                        pltpu.sync_copy(r1, u_h.at[pl.ds(cbase + base, ROW_TILE)])
                    return carry
                lax.fori_loop(0, ROWS_PER_TEC // ROW_TILE, ep, 0)
                plsc.subcore_barrier()
                return _
            lax.fori_loop(1, KMAX, k_body, 0)
            return _
        lax.fori_loop(0, CH, chunk_body, 0)

    return layer


def _matmul_tc(C_in, W_in, dout3, relu):
    M = 512
    KC = KMAX * C_in
    grid = (NP // M, KC)

    def mm(tx_ref, w_ref, b_ref, o_ref, accs):
        kci = pl.program_id(1)

        @pl.when(kci == 0)
        def _():
            accs[...] = jnp.zeros_like(accs)

        accs[...] += jnp.dot(tx_ref[0, 0], w_ref[0],
                             preferred_element_type=jnp.float32)

        @pl.when(kci == KC - 1)
        def _():
            r = accs[...] + b_ref[...]
            if relu:
                r = jnp.maximum(r, 0.0)
            o_ref[...] = r

    return pl.pallas_call(
        mm,
        grid=grid,
        in_specs=[
            pl.BlockSpec((1, 1, M, W_in),
                         lambda n, kci: (kci // C_in, kci % C_in, n, 0)),
            pl.BlockSpec((1, W_in, dout3), lambda n, kci: (kci, 0, 0)),
            pl.BlockSpec((1, dout3), lambda n, kci: (0, 0)),
        ],
        out_specs=pl.BlockSpec((M, dout3), lambda n, kci: (n, 0)),
        out_shape=jax.ShapeDtypeStruct((NP, dout3), jnp.float32),
        scratch_shapes=[pltpu.VMEM((M, dout3), jnp.float32)],
        compiler_params=pltpu.CompilerParams(
            dimension_semantics=("parallel", "arbitrary")),
    )


def _pack_weights(ws, bs, C_in, W_in):
    din = C_in * W_in
    dout = ws[0].shape[-1]
    dout3 = 3 * dout
    wf = jnp.zeros((KMAX, din, dout3), jnp.float32)
    for s, w in enumerate(ws):
        k = w.shape[0]
        wf = wf.at[:k, :, s * dout:(s + 1) * dout].set(w)
    wf = wf.reshape(KMAX * C_in, W_in, dout3)
    bf = jnp.concatenate(bs).reshape(1, dout3)
    return wf, bf


_CFG = [(2, 64), (2, 96), (4, 96), (8, 96)]


def kernel(x, edge_index, lambda_max,
           l1w0, l1b0, l1w1, l1b1, l1w2, l1b2,
           l2w0, l2b0, l2w1, l2b1, l2w2, l2b2,
           l3w0, l3b0, l3w1, l3b1, l3w2, l3b2,
           l4w0, l4b0, l4w1, l4b1, l4w2, l4b2,
           fcw, fcb):
    row = edge_index[0]
    col = edge_index[1]
    lam16 = jnp.broadcast_to(lambda_max.astype(jnp.float32), (16,))

    rowp, g = _setup_sc()(row, col, lam16)
    col3 = col.reshape(E_EDGES // EDGE_B, 1, EDGE_B)

    h = jnp.zeros((NP, 128), jnp.float32).at[:N_NODES].set(x)

    wss = [(l1w0, l1w1, l1w2), (l2w0, l2w1, l2w2),
           (l3w0, l3w1, l3w2), (l4w0, l4w1, l4w2)]
    bss = [(l1b0, l1b1, l1b2), (l2b0, l2b1, l2b2),
           (l3b0, l3b1, l3b2), (l4b0, l4b1, l4b2)]

    for li in range(4):
        C_in, W_in = _CFG[li]
        hc = h.reshape(NP, C_in, W_in).transpose(1, 0, 2)
        tx, _u = _layer_sc(C_in, W_in)(hc, rowp, col3, g, lam16)
        wf, bf = _pack_weights(wss[li], bss[li], C_in, W_in)
        dout3 = bf.shape[-1]
        h = _matmul_tc(C_in, W_in, dout3, relu=True)(tx, wf, bf)

    M = 512

    def fc(h_ref, w_ref, b_ref, o_ref):
        o_ref[...] = jnp.dot(h_ref[...], w_ref[...],
                             preferred_element_type=jnp.float32) + b_ref[...]

    out = pl.pallas_call(
        fc,
        grid=(NP // M,),
        in_specs=[
            pl.BlockSpec((M, 768), lambda n: (n, 0)),
            pl.BlockSpec((768, 16), lambda n: (0, 0)),
            pl.BlockSpec((1, 16), lambda n: (0, 0)),
        ],
        out_specs=pl.BlockSpec((M, 16), lambda n: (n, 0)),
        out_shape=jax.ShapeDtypeStruct((NP, 16), jnp.float32),
        compiler_params=pltpu.CompilerParams(
            dimension_semantics=("parallel",)),
    )(h, fcw, fcb.reshape(1, 16))

    return out[:N_NODES]
